# baseline (speedup 1.0000x reference)
import functools

import jax
import jax.numpy as jnp
from jax import lax
from jax.experimental import pallas as pl
from jax.experimental.pallas import tpu as pltpu
from jax.experimental.pallas import tpu_sc as plsc

N = 8388608
NC = 2
NS = 16
L = 16
NW = NC * NS
CHUNK = 24576

SC_CHUNKS = 4
PER_W = CHUNK * SC_CHUNKS
N_SC = NW * PER_W

LANES = 128
ROWS = N // LANES
TR0 = N_SC // LANES
BR = 8192
GT = (ROWS - TR0) // BR

W_LO = 2.0 / 3.0


@functools.partial(
    pl.kernel,
    out_type=jax.ShapeDtypeStruct((NW, 2, L), jnp.float32),
    mesh=plsc.VectorSubcoreMesh(core_axis_name="c", subcore_axis_name="s",
                                num_cores=NC, num_subcores=NS),
    scratch_types=[
        pltpu.VMEM((CHUNK,), jnp.float32),
        pltpu.VMEM((CHUNK,), jnp.float32),
        pltpu.VMEM((CHUNK,), jnp.float32),
        pltpu.VMEM((CHUNK,), jnp.float32),
        pltpu.VMEM((2, L), jnp.float32),
        pltpu.SemaphoreType.DMA,
        pltpu.SemaphoreType.DMA,
    ],
)
def _sc_partials(pred_hbm, gt_hbm, out_hbm, p0, p1, g0, g1, stage, sem0, sem1):
    cid = lax.axis_index("c")
    sid = lax.axis_index("s")
    wid = sid * NC + cid
    base = wid * PER_W

    pbuf = (p0, p1)
    gbuf = (g0, g1)
    sems = (sem0, sem1)

    def start(slot, off):
        pltpu.async_copy(pred_hbm.at[pl.ds(off, CHUNK)], pbuf[slot], sems[slot])
        pltpu.async_copy(gt_hbm.at[pl.ds(off, CHUNK)], gbuf[slot], sems[slot])

    def wait(slot):
        pltpu.make_async_copy(pred_hbm.at[pl.ds(base, CHUNK)], pbuf[slot], sems[slot]).wait()
        pltpu.make_async_copy(gt_hbm.at[pl.ds(base, CHUNK)], gbuf[slot], sems[slot]).wait()

    U = 8

    def compute(pv, gv, acc):
        @pl.loop(0, CHUNK // L // U, init_carry=acc)
        def acc_loop(j, carry):
            aws = list(carry[:U])
            awds = list(carry[U:])
            base_i = j * (U * L)
            for u in range(U):
                p = pv[pl.ds(base_i + u * L, L)]
                g = gv[pl.ds(base_i + u * L, L)]
                d = g - p
                w = jnp.where(g <= 0.55, W_LO, 1.0)
                aws[u] = aws[u] + w
                awds[u] = awds[u] + w * (d * d)
            return tuple(aws) + tuple(awds)

        return acc_loop

    zero = jnp.zeros((L,), jnp.float32)
    acc = (zero,) * (2 * U)
    start(0, base)
    start(1, base + CHUNK)

    @pl.loop(0, SC_CHUNKS // 2 - 1, init_carry=acc)
    def chunk_loop(kk, acc):
        for b in range(2):
            k = 2 * kk + b
            wait(b)
            acc = compute(pbuf[b], gbuf[b], acc)
            start(b, base + (k + 2) * CHUNK)
        return acc

    acc = chunk_loop
    for b in range(2):
        wait(b)
        acc = compute(pbuf[b], gbuf[b], acc)

    def _tree(vs):
        vs = list(vs)
        while len(vs) > 1:
            vs = [a + b for a, b in zip(vs[::2], vs[1::2])]
        return vs[0]

    stage[0, :] = _tree(acc[:U])
    stage[1, :] = _tree(acc[U:])
    pltpu.sync_copy(stage, out_hbm.at[wid])


def _tc_body(p_ref, g_ref, out_ref, acc_ref):
    i = pl.program_id(0)

    @pl.when(i == 0)
    def _init():
        acc_ref[...] = jnp.zeros_like(acc_ref)

    p = p_ref[...]
    g = g_ref[...]
    d = g - p
    w = jnp.where(g <= 0.55, W_LO, 1.0)
    sw = jnp.sum(w.reshape(BR // 8, 8, LANES), axis=0)
    swd = jnp.sum((w * (d * d)).reshape(BR // 8, 8, LANES), axis=0)
    acc_ref[0] += sw
    acc_ref[1] += swd

    @pl.when(i == GT - 1)
    def _emit():
        out_ref[...] = acc_ref[...]


_tc_partials = pl.pallas_call(
    _tc_body,
    grid=(GT,),
    in_specs=[
        pl.BlockSpec((BR, LANES), lambda i: (TR0 // BR + i, 0)),
        pl.BlockSpec((BR, LANES), lambda i: (TR0 // BR + i, 0)),
    ],
    out_specs=pl.BlockSpec((2, 8, LANES), lambda i: (0, 0, 0)),
    out_shape=jax.ShapeDtypeStruct((2, 8, LANES), jnp.float32),
    scratch_shapes=[pltpu.VMEM((2, 8, LANES), jnp.float32)],
)


def kernel(y_pred, y_gt):
    sc = _sc_partials(y_pred, y_gt)
    p2 = y_pred.reshape(ROWS, LANES)
    g2 = y_gt.reshape(ROWS, LANES)
    tc = _tc_partials(p2, g2)
    sc_sums = jnp.sum(sc, axis=(0, 2))
    tc_sums = jnp.sum(tc, axis=(1, 2))
    sum_w = sc_sums[0] + tc_sums[0]
    sum_wd = sc_sums[1] + tc_sums[1]
    return sum_wd / sum_w


# speedup vs baseline: 1.1161x; 1.1161x over previous
import functools

import jax
import jax.numpy as jnp
from jax import lax
from jax.experimental import pallas as pl
from jax.experimental.pallas import tpu as pltpu
from jax.experimental.pallas import tpu_sc as plsc

N = 8388608
NC = 2
NS = 16
L = 16
NW = NC * NS
CHUNK = 12288

SC_CHUNKS = 8
PER_W = CHUNK * SC_CHUNKS
N_SC = NW * PER_W

LANES = 128
ROWS = N // LANES
TR0 = N_SC // LANES
BR = 8192
GT = (ROWS - TR0) // BR

W_LO = 2.0 / 3.0


@functools.partial(
    pl.kernel,
    out_type=jax.ShapeDtypeStruct((NW, 2, L), jnp.float32),
    mesh=plsc.VectorSubcoreMesh(core_axis_name="c", subcore_axis_name="s",
                                num_cores=NC, num_subcores=NS),
    scratch_types=[
        pltpu.VMEM((CHUNK,), jnp.float32),
        pltpu.VMEM((CHUNK,), jnp.float32),
        pltpu.VMEM((CHUNK,), jnp.float32),
        pltpu.VMEM((CHUNK,), jnp.float32),
        pltpu.VMEM((2, L), jnp.float32),
        pltpu.SemaphoreType.DMA,
        pltpu.SemaphoreType.DMA,
    ],
)
def _sc_partials(pred_hbm, gt_hbm, out_hbm, p0, p1, g0, g1, stage, sem0, sem1):
    cid = lax.axis_index("c")
    sid = lax.axis_index("s")
    wid = sid * NC + cid
    base = wid * PER_W

    pbuf = (p0, p1)
    gbuf = (g0, g1)
    sems = (sem0, sem1)

    def start(slot, off):
        pltpu.async_copy(pred_hbm.at[pl.ds(off, CHUNK)], pbuf[slot], sems[slot])
        pltpu.async_copy(gt_hbm.at[pl.ds(off, CHUNK)], gbuf[slot], sems[slot])

    def wait(slot):
        pltpu.make_async_copy(pred_hbm.at[pl.ds(base, CHUNK)], pbuf[slot], sems[slot]).wait()
        pltpu.make_async_copy(gt_hbm.at[pl.ds(base, CHUNK)], gbuf[slot], sems[slot]).wait()

    U = 8

    def compute(pv, gv, acc):
        @pl.loop(0, CHUNK // L // U, init_carry=acc)
        def acc_loop(j, carry):
            aws = list(carry[:U])
            awds = list(carry[U:])
            base_i = j * (U * L)
            for u in range(U):
                p = pv[pl.ds(base_i + u * L, L)]
                g = gv[pl.ds(base_i + u * L, L)]
                d = g - p
                w = jnp.where(g <= 0.55, W_LO, 1.0)
                aws[u] = aws[u] + w
                awds[u] = awds[u] + w * (d * d)
            return tuple(aws) + tuple(awds)

        return acc_loop

    zero = jnp.zeros((L,), jnp.float32)
    acc = (zero,) * (2 * U)
    start(0, base)
    start(1, base + CHUNK)

    @pl.loop(0, SC_CHUNKS // 2 - 1, init_carry=acc)
    def chunk_loop(kk, acc):
        for b in range(2):
            k = 2 * kk + b
            wait(b)
            acc = compute(pbuf[b], gbuf[b], acc)
            start(b, base + (k + 2) * CHUNK)
        return acc

    acc = chunk_loop
    for b in range(2):
        wait(b)
        acc = compute(pbuf[b], gbuf[b], acc)

    def _tree(vs):
        vs = list(vs)
        while len(vs) > 1:
            vs = [a + b for a, b in zip(vs[::2], vs[1::2])]
        return vs[0]

    stage[0, :] = _tree(acc[:U])
    stage[1, :] = _tree(acc[U:])
    pltpu.sync_copy(stage, out_hbm.at[wid])


def _tc_body(p_ref, g_ref, out_ref, acc_ref):
    i = pl.program_id(0)

    @pl.when(i == 0)
    def _init():
        acc_ref[...] = jnp.zeros_like(acc_ref)

    p = p_ref[...]
    g = g_ref[...]
    d = g - p
    w = jnp.where(g <= 0.55, W_LO, 1.0)
    sw = jnp.sum(w.reshape(BR // 8, 8, LANES), axis=0)
    swd = jnp.sum((w * (d * d)).reshape(BR // 8, 8, LANES), axis=0)
    acc_ref[0] += sw
    acc_ref[1] += swd

    @pl.when(i == GT - 1)
    def _emit():
        out_ref[...] = acc_ref[...]


_tc_partials = pl.pallas_call(
    _tc_body,
    grid=(GT,),
    in_specs=[
        pl.BlockSpec((BR, LANES), lambda i: (TR0 // BR + i, 0)),
        pl.BlockSpec((BR, LANES), lambda i: (TR0 // BR + i, 0)),
    ],
    out_specs=pl.BlockSpec((2, 8, LANES), lambda i: (0, 0, 0)),
    out_shape=jax.ShapeDtypeStruct((2, 8, LANES), jnp.float32),
    scratch_shapes=[pltpu.VMEM((2, 8, LANES), jnp.float32)],
)


def _combine_body(sc_ref, tc_ref, out_ref):
    sum_w = jnp.sum(sc_ref[:, 0, :]) + jnp.sum(tc_ref[0])
    sum_wd = jnp.sum(sc_ref[:, 1, :]) + jnp.sum(tc_ref[1])
    out_ref[...] = (sum_wd / sum_w)[None, None]


_combine = pl.pallas_call(
    _combine_body,
    out_shape=jax.ShapeDtypeStruct((1, 1), jnp.float32),
)


def kernel(y_pred, y_gt):
    sc = _sc_partials(y_pred, y_gt)
    p2 = y_pred.reshape(ROWS, LANES)
    g2 = y_gt.reshape(ROWS, LANES)
    tc = _tc_partials(p2, g2)
    return _combine(sc, tc)[0, 0]


# speedup vs baseline: 1.1324x; 1.0146x over previous
import functools

import jax
import jax.numpy as jnp
from jax import lax
from jax.experimental import pallas as pl
from jax.experimental.pallas import tpu as pltpu
from jax.experimental.pallas import tpu_sc as plsc

N = 8388608
NC = 2
NS = 16
L = 16
NW = NC * NS
CHUNK = 8192

SC_CHUNKS = 8
PER_W = CHUNK * SC_CHUNKS
N_SC = NW * PER_W

LANES = 128
ROWS = N // LANES
TR0 = N_SC // LANES
BR = 8192
GT = (ROWS - TR0) // BR

W_LO = 2.0 / 3.0


@functools.partial(
    pl.kernel,
    out_type=jax.ShapeDtypeStruct((NW, 2, L), jnp.float32),
    mesh=plsc.VectorSubcoreMesh(core_axis_name="c", subcore_axis_name="s",
                                num_cores=NC, num_subcores=NS),
    scratch_types=[
        pltpu.VMEM((CHUNK,), jnp.float32),
        pltpu.VMEM((CHUNK,), jnp.float32),
        pltpu.VMEM((CHUNK,), jnp.float32),
        pltpu.VMEM((CHUNK,), jnp.float32),
        pltpu.VMEM((2, L), jnp.float32),
        pltpu.SemaphoreType.DMA,
        pltpu.SemaphoreType.DMA,
    ],
)
def _sc_partials(pred_hbm, gt_hbm, out_hbm, p0, p1, g0, g1, stage, sem0, sem1):
    cid = lax.axis_index("c")
    sid = lax.axis_index("s")
    wid = sid * NC + cid
    base = wid * PER_W

    pbuf = (p0, p1)
    gbuf = (g0, g1)
    sems = (sem0, sem1)

    def start(slot, off):
        pltpu.async_copy(pred_hbm.at[pl.ds(off, CHUNK)], pbuf[slot], sems[slot])
        pltpu.async_copy(gt_hbm.at[pl.ds(off, CHUNK)], gbuf[slot], sems[slot])

    def wait(slot):
        pltpu.make_async_copy(pred_hbm.at[pl.ds(base, CHUNK)], pbuf[slot], sems[slot]).wait()
        pltpu.make_async_copy(gt_hbm.at[pl.ds(base, CHUNK)], gbuf[slot], sems[slot]).wait()

    U = 8

    def compute(pv, gv, acc):
        @pl.loop(0, CHUNK // L // U, init_carry=acc)
        def acc_loop(j, carry):
            aws = list(carry[:U])
            awds = list(carry[U:])
            base_i = j * (U * L)
            for u in range(U):
                p = pv[pl.ds(base_i + u * L, L)]
                g = gv[pl.ds(base_i + u * L, L)]
                d = g - p
                w = jnp.where(g <= 0.55, W_LO, 1.0)
                aws[u] = aws[u] + w
                awds[u] = awds[u] + w * (d * d)
            return tuple(aws) + tuple(awds)

        return acc_loop

    zero = jnp.zeros((L,), jnp.float32)
    acc = (zero,) * (2 * U)
    start(0, base)
    start(1, base + CHUNK)

    @pl.loop(0, SC_CHUNKS // 2 - 1, init_carry=acc)
    def chunk_loop(kk, acc):
        for b in range(2):
            k = 2 * kk + b
            wait(b)
            acc = compute(pbuf[b], gbuf[b], acc)
            start(b, base + (k + 2) * CHUNK)
        return acc

    acc = chunk_loop
    for b in range(2):
        wait(b)
        acc = compute(pbuf[b], gbuf[b], acc)

    def _tree(vs):
        vs = list(vs)
        while len(vs) > 1:
            vs = [a + b for a, b in zip(vs[::2], vs[1::2])]
        return vs[0]

    stage[0, :] = _tree(acc[:U])
    stage[1, :] = _tree(acc[U:])
    pltpu.sync_copy(stage, out_hbm.at[wid])


def _tc_body(p_ref, g_ref, out_ref, acc_ref):
    i = pl.program_id(0)

    @pl.when(i == 0)
    def _init():
        acc_ref[...] = jnp.zeros_like(acc_ref)

    p = p_ref[...]
    g = g_ref[...]
    d = g - p
    w = jnp.where(g <= 0.55, W_LO, 1.0)
    sw = jnp.sum(w.reshape(BR // 8, 8, LANES), axis=0)
    swd = jnp.sum((w * (d * d)).reshape(BR // 8, 8, LANES), axis=0)
    acc_ref[0] += sw
    acc_ref[1] += swd

    @pl.when(i == GT - 1)
    def _emit():
        out_ref[...] = acc_ref[...]


_tc_partials = pl.pallas_call(
    _tc_body,
    grid=(GT,),
    in_specs=[
        pl.BlockSpec((BR, LANES), lambda i: (TR0 // BR + i, 0)),
        pl.BlockSpec((BR, LANES), lambda i: (TR0 // BR + i, 0)),
    ],
    out_specs=pl.BlockSpec((2, 8, LANES), lambda i: (0, 0, 0)),
    out_shape=jax.ShapeDtypeStruct((2, 8, LANES), jnp.float32),
    scratch_shapes=[pltpu.VMEM((2, 8, LANES), jnp.float32)],
)


def _combine_body(sc_ref, tc_ref, out_ref):
    sum_w = jnp.sum(sc_ref[:, 0, :]) + jnp.sum(tc_ref[0])
    sum_wd = jnp.sum(sc_ref[:, 1, :]) + jnp.sum(tc_ref[1])
    out_ref[...] = (sum_wd / sum_w)[None, None]


_combine = pl.pallas_call(
    _combine_body,
    out_shape=jax.ShapeDtypeStruct((1, 1), jnp.float32),
)


def kernel(y_pred, y_gt):
    sc = _sc_partials(y_pred, y_gt)
    p2 = y_pred.reshape(ROWS, LANES)
    g2 = y_gt.reshape(ROWS, LANES)
    tc = _tc_partials(p2, g2)
    return _combine(sc, tc)[0, 0]
